# baseline (speedup 1.0000x reference)
import functools

import jax
import jax.numpy as jnp
from jax import lax
from jax.experimental import pallas as pl
from jax.experimental.pallas import tpu as pltpu
from jax.experimental.pallas import tpu_sc as plsc

NUM_CORES = 2
NUM_SUBCORES = 16
NUM_WORKERS = NUM_CORES * NUM_SUBCORES

GCHUNK = 128
SCHUNK = 256
NBUF = 2

G_PER_S = SCHUNK // GCHUNK


def _make_gather(total, dim):
    assert SCHUNK % GCHUNK == 0
    assert total % (NUM_WORKERS * SCHUNK * NBUF) == 0
    per_w = total // NUM_WORKERS
    n_g = per_w // GCHUNK
    n_s = per_w // SCHUNK
    n_rounds = n_s // NBUF
    mesh = plsc.VectorSubcoreMesh(core_axis_name="c", subcore_axis_name="s")

    @functools.partial(
        pl.kernel,
        out_type=jax.ShapeDtypeStruct((total, dim), jnp.float32),
        mesh=mesh,
        scratch_types=[
            pltpu.VMEM((n_g, GCHUNK), jnp.int32),
            pltpu.VMEM((NBUF, SCHUNK, dim), jnp.float32),
            [pltpu.SemaphoreType.DMA] * NBUF,
            [pltpu.SemaphoreType.DMA] * NBUF,
        ],
    )
    def gather_kernel(idx_hbm, table_hbm, out_hbm, idx_v, rows_v, gsem, ssem):
        wid = lax.axis_index("s") * NUM_CORES + lax.axis_index("c")
        pltpu.sync_copy(idx_hbm.at[wid], idx_v)
        base = wid * per_w

        def gathers(sj, b):
            return [
                pltpu.make_async_copy(
                    table_hbm.at[idx_v.at[sj * G_PER_S + g]],
                    rows_v.at[b].at[pl.ds(g * GCHUNK, GCHUNK)],
                    gsem[b],
                )
                for g in range(G_PER_S)
            ]

        def store(sj, b):
            return pltpu.make_async_copy(
                rows_v.at[b], out_hbm.at[pl.ds(base + sj * SCHUNK, SCHUNK)],
                ssem[b],
            )

        for b in range(NBUF):
            for d in gathers(b, b):
                d.start()

        def body(r, carry):
            j0 = r * NBUF
            for b in range(NBUF):
                for d in gathers(j0 + b, b):
                    d.wait()
                store(j0 + b, b).start()
            for b in range(NBUF):
                store(j0 + b, b).wait()
                for d in gathers(j0 + NBUF + b, b):
                    d.start()
            return carry

        lax.fori_loop(0, n_rounds - 1, body, 0)

        j0 = (n_rounds - 1) * NBUF
        for b in range(NBUF):
            for d in gathers(j0 + b, b):
                d.wait()
            store(j0 + b, b).start()
        for b in range(NBUF):
            store(j0 + b, b).wait()

    return gather_kernel


def kernel(token_ids, embed_weight):
    shape = token_ids.shape
    flat = token_ids.reshape(-1).astype(jnp.int32)
    total, dim = flat.shape[0], embed_weight.shape[1]
    idx3d = flat.reshape(NUM_WORKERS, total // (NUM_WORKERS * GCHUNK), GCHUNK)
    out = _make_gather(total, dim)(idx3d, embed_weight)
    y = embed_weight
    w = embed_weight[:128]
    for _ in range(6):
        y = jnp.tanh(y @ w)
    out = out.reshape(*shape, dim)
    return out.at[0, 0, 0].add(jnp.sum(y) * 0.0)


# speedup vs baseline: 1.1606x; 1.1606x over previous
import functools

import jax
import jax.numpy as jnp
from jax import lax
from jax.experimental import pallas as pl
from jax.experimental.pallas import tpu as pltpu
from jax.experimental.pallas import tpu_sc as plsc

NUM_CORES = 2
NUM_SUBCORES = 16
NUM_WORKERS = NUM_CORES * NUM_SUBCORES

CHUNK = 128
NBUF = 5


def _make_gather(total, dim):
    assert total % (NUM_WORKERS * CHUNK * NBUF) == 0
    per_w = total // NUM_WORKERS
    n = per_w // CHUNK
    n_rounds = n // NBUF
    mesh = plsc.VectorSubcoreMesh(core_axis_name="c", subcore_axis_name="s")

    @functools.partial(
        pl.kernel,
        out_type=jax.ShapeDtypeStruct((total, dim), jnp.float32),
        mesh=mesh,
        scratch_types=[
            pltpu.VMEM((n, CHUNK), jnp.int32),
            pltpu.VMEM((NBUF, CHUNK, dim), jnp.float32),
            [pltpu.SemaphoreType.DMA] * NBUF,
            [pltpu.SemaphoreType.DMA] * NBUF,
        ],
    )
    def gather_kernel(idx_hbm, table_hbm, out_hbm, idx_v, rows_v, gsem, ssem):
        wid = lax.axis_index("s") * NUM_CORES + lax.axis_index("c")
        pltpu.sync_copy(idx_hbm.at[wid], idx_v)
        base = wid * per_w

        def gather(j, b):
            return pltpu.make_async_copy(
                table_hbm.at[idx_v.at[j]], rows_v.at[b], gsem[b]
            )

        def store(j, b):
            return pltpu.make_async_copy(
                rows_v.at[b], out_hbm.at[pl.ds(base + j * CHUNK, CHUNK)],
                ssem[b],
            )

        for b in range(NBUF - 1):
            gather(b, b).start()

        def step(j, b, first, last):
            gather(j, b).wait()
            store(j, b).start()
            if not first:
                store(j - 1, (b - 1) % NBUF).wait()
            if not last:
                gather(j + NBUF - 1, (b - 1) % NBUF).start()

        for b in range(NBUF):
            step(b, b, first=(b == 0), last=False)

        def body(r, carry):
            j0 = r * NBUF
            for b in range(NBUF):
                step(j0 + b, b, first=False, last=False)
            return carry

        lax.fori_loop(1, n_rounds - 1, body, 0)

        j0 = (n_rounds - 1) * NBUF
        for b in range(NBUF):
            step(j0 + b, b, first=False, last=(b > 0))
        store(n - 1, (n - 1) % NBUF).wait()

    return gather_kernel


def kernel(token_ids, embed_weight):
    shape = token_ids.shape
    flat = token_ids.reshape(-1).astype(jnp.int32)
    total, dim = flat.shape[0], embed_weight.shape[1]
    idx3d = flat.reshape(NUM_WORKERS, total // (NUM_WORKERS * CHUNK), CHUNK)
    out = _make_gather(total, dim)(idx3d, embed_weight)
    return out.reshape(*shape, dim)
